# baseline (speedup 1.0000x reference)
import functools

import jax
import jax.numpy as jnp
from jax import lax
from jax.experimental import pallas as pl
from jax.experimental.pallas import tpu as pltpu
from jax.experimental.pallas import tpu_sc as plsc

_NTASK = 1000
_TPAD = 1024
_E = 64
_B = 16384
_K = 8
_PREC = lax.Precision.DEFAULT


def _fused_gate_pipeline(x, w0, b0, w1, b1, w2, b2, bn0=256, bn1=256):
    m, d = x.shape
    hid = w0.shape[1]
    e = w2.shape[1]
    nj0 = hid // bn0
    nj1 = hid // bn1
    steps = nj0 + nj1 + 1

    d2 = d // 2
    h2 = hid // 2

    def body(x_ref, w0a_ref, w0b_ref, b0_ref, w1a_ref, w1b_ref, b1_ref,
             w2_ref, b2_ref, o_ref, h0_ref, h1_ref):
        s = pl.program_id(0)

        @pl.when(s < nj0)
        def _():
            js = pl.ds(s * bn0, bn0)
            h0_ref[:, js] = jnp.maximum(
                jnp.dot(x_ref[:, :d2], w0a_ref[...],
                        preferred_element_type=jnp.float32, precision=_PREC)
                + jnp.dot(x_ref[:, d2:], w0b_ref[...],
                          preferred_element_type=jnp.float32, precision=_PREC)
                + b0_ref[:, js], 0.0)

        @pl.when(jnp.logical_and(s >= nj0, s < nj0 + nj1))
        def _():
            js = pl.ds((s - nj0) * bn1, bn1)
            h1_ref[:, js] = jnp.maximum(
                jnp.dot(h0_ref[:, :h2], w1a_ref[...],
                        preferred_element_type=jnp.float32, precision=_PREC)
                + jnp.dot(h0_ref[:, h2:], w1b_ref[...],
                          preferred_element_type=jnp.float32, precision=_PREC)
                + b1_ref[:, js], 0.0)

        @pl.when(s == nj0 + nj1)
        def _():
            logits = jnp.dot(h1_ref[...], w2_ref[...],
                             preferred_element_type=jnp.float32,
                             precision=_PREC) + b2_ref[...]
            mx = jnp.max(logits, axis=1, keepdims=True)
            eg = jnp.exp(logits - mx)
            gate = eg / jnp.sum(eg, axis=1, keepdims=True)
            col = lax.broadcasted_iota(jnp.int32, (m, e), 1)
            work = gate
            mask = jnp.zeros((m, e), jnp.bool_)
            for _ in range(_K):
                wmx = jnp.max(work, axis=1, keepdims=True)
                cand = jnp.where(work == wmx, col, e)
                first = jnp.min(cand, axis=1, keepdims=True)
                sel = col == first
                mask = jnp.logical_or(mask, sel)
                work = jnp.where(sel, -jnp.inf, work)
            g = jnp.where(mask, gate, 0.0)
            g = g / jnp.sum(g, axis=1, keepdims=True)
            o_ref[...] = g.T

    clamp = lambda v, hi_: jnp.clip(v, 0, hi_)
    return pl.pallas_call(
        body,
        grid=(steps,),
        in_specs=[
            pl.BlockSpec((m, d), lambda s: (0, 0)),
            pl.BlockSpec((d2, bn0), lambda s: (0, clamp(s, nj0 - 1))),
            pl.BlockSpec((d2, bn0), lambda s: (1, clamp(s, nj0 - 1))),
            pl.BlockSpec((1, hid), lambda s: (0, 0)),
            pl.BlockSpec((h2, bn1), lambda s: (0, clamp(s - nj0, nj1 - 1))),
            pl.BlockSpec((h2, bn1), lambda s: (1, clamp(s - nj0, nj1 - 1))),
            pl.BlockSpec((1, hid), lambda s: (0, 0)),
            pl.BlockSpec((hid, e), lambda s: (0, 0)),
            pl.BlockSpec((1, e), lambda s: (0, 0)),
        ],
        out_specs=pl.BlockSpec((e, m), lambda s: (0, 0)),
        out_shape=jax.ShapeDtypeStruct((e, m), jnp.float32),
        scratch_shapes=[
            pltpu.VMEM((m, hid), jnp.float32),
            pltpu.VMEM((m, hid), jnp.float32),
        ],
        compiler_params=pltpu.CompilerParams(
            dimension_semantics=("arbitrary",)),
    )(x, w0, w0, b0, w1, w1, b1, w2, b2)


def _sc_expand(gate_t, idx):
    info = plsc.get_sparse_core_info()
    nc, ns = info.num_cores, info.num_subcores
    nw = nc * ns
    e_per_w = _E // nw
    mesh = plsc.VectorSubcoreMesh(core_axis_name="c", subcore_axis_name="s")

    @functools.partial(
        pl.kernel,
        mesh=mesh,
        out_type=jax.ShapeDtypeStruct((_E, _B), jnp.float32),
        scratch_types=(
            [pltpu.VMEM((_B,), jnp.int32)]
            + [pltpu.VMEM((_B,), jnp.float32) for _ in range(e_per_w)]
            + [pltpu.VMEM((_TPAD,), jnp.float32) for _ in range(e_per_w)]
            + [pltpu.SemaphoreType.DMA, pltpu.SemaphoreType.DMA]
        ),
        compiler_params=pltpu.CompilerParams(needs_layout_passes=False),
    )
    def k(gate_hbm, idx_hbm, out_hbm, idx_v, *rest):
        out_bufs = rest[:e_per_w]
        row_bufs = rest[e_per_w:2 * e_per_w]
        sem_in, sem_out = rest[2 * e_per_w], rest[2 * e_per_w + 1]
        wid = lax.axis_index("s") * nc + lax.axis_index("c")
        e_base = wid * e_per_w
        in_cps = [pltpu.async_copy(idx_hbm, idx_v, sem_in)]
        for j in range(e_per_w):
            in_cps.append(
                pltpu.async_copy(gate_hbm.at[e_base + j], row_bufs[j],
                                 sem_in))
        for cp in in_cps:
            cp.wait()
        out_cps = []
        for j in range(e_per_w):
            row_v = row_bufs[j]
            out_v = out_bufs[j]

            def body(i, _):
                for u in range(8):
                    sl = pl.ds(i * 128 + u * 16, 16)
                    out_v[sl] = plsc.load_gather(row_v, [idx_v[sl]])
                return 0

            lax.fori_loop(0, _B // 128, body, 0)
            out_cps.append(
                pltpu.async_copy(out_v, out_hbm.at[e_base + j], sem_out))
        for cp in out_cps:
            cp.wait()

    return k(gate_t, idx)


def kernel(env_index, emb_table, W0, b0, W1, b1, W2, b2):
    x = jnp.pad(emb_table, ((0, _TPAD - _NTASK), (0, 0)))
    gate_t = _fused_gate_pipeline(x, W0, b0.reshape(1, -1), W1,
                                  b1.reshape(1, -1), W2, b2.reshape(1, -1))
    out_t = _sc_expand(gate_t, env_index.astype(jnp.int32))
    return out_t[:, :, None]


# speedup vs baseline: 1.0784x; 1.0784x over previous
import functools

import jax
import jax.numpy as jnp
from jax import lax
from jax.experimental import pallas as pl
from jax.experimental.pallas import tpu as pltpu
from jax.experimental.pallas import tpu_sc as plsc

_NTASK = 1000
_TPAD = 1024
_E = 64
_B = 16384
_K = 8
_PREC = lax.Precision.DEFAULT


def _fused_gate_pipeline(x, w0, b0, w1, b1, w2, b2, bn0=256, bn1=256):
    m, d = x.shape
    hid = w0.shape[1]
    e = w2.shape[1]
    nj0 = hid // bn0
    nj1 = hid // bn1
    steps = nj0 + nj1 + 1

    def body(x_ref, w0_ref, b0_ref, w1_ref, b1_ref, w2_ref, b2_ref, o_ref,
             h0_ref, h1_ref):
        s = pl.program_id(0)

        @pl.when(s < nj0)
        def _():
            js = pl.ds(s * bn0, bn0)
            h0_ref[:, js] = jnp.maximum(
                jnp.dot(x_ref[...], w0_ref[...],
                        preferred_element_type=jnp.float32, precision=_PREC)
                + b0_ref[:, js], 0.0)

        @pl.when(jnp.logical_and(s >= nj0, s < nj0 + nj1))
        def _():
            js = pl.ds((s - nj0) * bn1, bn1)
            h1_ref[:, js] = jnp.maximum(
                jnp.dot(h0_ref[...], w1_ref[...],
                        preferred_element_type=jnp.float32, precision=_PREC)
                + b1_ref[:, js], 0.0)

        @pl.when(s == nj0 + nj1)
        def _():
            logits = jnp.dot(h1_ref[...], w2_ref[...],
                             preferred_element_type=jnp.float32,
                             precision=_PREC) + b2_ref[...]
            mx = jnp.max(logits, axis=1, keepdims=True)
            eg = jnp.exp(logits - mx)
            gate = eg / jnp.sum(eg, axis=1, keepdims=True)
            col = lax.broadcasted_iota(jnp.int32, (m, e), 1)
            work = gate
            mask = jnp.zeros((m, e), jnp.bool_)
            for _ in range(_K):
                wmx = jnp.max(work, axis=1, keepdims=True)
                cand = jnp.where(work == wmx, col, e)
                first = jnp.min(cand, axis=1, keepdims=True)
                sel = col == first
                mask = jnp.logical_or(mask, sel)
                work = jnp.where(sel, -jnp.inf, work)
            g = jnp.where(mask, gate, 0.0)
            g = g / jnp.sum(g, axis=1, keepdims=True)
            o_ref[...] = g.T

    clamp = lambda v, hi_: jnp.clip(v, 0, hi_)
    return pl.pallas_call(
        body,
        grid=(steps,),
        in_specs=[
            pl.BlockSpec((m, d), lambda s: (0, 0)),
            pl.BlockSpec((d, bn0), lambda s: (0, clamp(s, nj0 - 1))),
            pl.BlockSpec((1, hid), lambda s: (0, 0)),
            pl.BlockSpec((hid, bn1), lambda s: (0, clamp(s - nj0, nj1 - 1))),
            pl.BlockSpec((1, hid), lambda s: (0, 0)),
            pl.BlockSpec((hid, e), lambda s: (0, 0)),
            pl.BlockSpec((1, e), lambda s: (0, 0)),
        ],
        out_specs=pl.BlockSpec((e, m), lambda s: (0, 0)),
        out_shape=jax.ShapeDtypeStruct((e, m), jnp.float32),
        scratch_shapes=[
            pltpu.VMEM((m, hid), jnp.float32),
            pltpu.VMEM((m, hid), jnp.float32),
        ],
        compiler_params=pltpu.CompilerParams(
            dimension_semantics=("arbitrary",)),
    )(x, w0, b0, w1, b1, w2, b2)


def _sc_expand(gate_t, idx):
    info = plsc.get_sparse_core_info()
    nc, ns = info.num_cores, info.num_subcores
    nw = nc * ns
    e_per_w = _E // nw
    mesh = plsc.VectorSubcoreMesh(core_axis_name="c", subcore_axis_name="s")

    @functools.partial(
        pl.kernel,
        mesh=mesh,
        out_type=jax.ShapeDtypeStruct((_E, _B), jnp.float32),
        scratch_types=(
            [pltpu.VMEM((_B,), jnp.int32)]
            + [pltpu.VMEM((_B,), jnp.float32) for _ in range(e_per_w)]
            + [pltpu.VMEM((_NTASK,), jnp.float32) for _ in range(e_per_w)]
            + [pltpu.SemaphoreType.DMA, pltpu.SemaphoreType.DMA]
        ),
        compiler_params=pltpu.CompilerParams(needs_layout_passes=False),
    )
    def k(gate_hbm, idx_hbm, out_hbm, idx_v, *rest):
        out_bufs = rest[:e_per_w]
        row_bufs = rest[e_per_w:2 * e_per_w]
        sem_in, sem_out = rest[2 * e_per_w], rest[2 * e_per_w + 1]
        wid = lax.axis_index("s") * nc + lax.axis_index("c")
        e_base = wid * e_per_w
        in_cps = [pltpu.async_copy(idx_hbm, idx_v, sem_in)]
        for j in range(e_per_w):
            in_cps.append(
                pltpu.async_copy(gate_hbm.at[e_base + j], row_bufs[j],
                                 sem_in))
        for cp in in_cps:
            cp.wait()
        out_cps = []
        for j in range(e_per_w):
            row_v = row_bufs[j]
            out_v = out_bufs[j]

            def body(i, _):
                for u in range(8):
                    sl = pl.ds(i * 128 + u * 16, 16)
                    out_v[sl] = plsc.load_gather(row_v, [idx_v[sl]])
                return 0

            lax.fori_loop(0, _B // 128, body, 0)
            out_cps.append(
                pltpu.async_copy(out_v, out_hbm.at[e_base + j], sem_out))
        for cp in out_cps:
            cp.wait()

    return k(gate_t, idx)


def kernel(env_index, emb_table, W0, b0, W1, b1, W2, b2):
    gate_t = _fused_gate_pipeline(emb_table, W0, b0.reshape(1, -1), W1,
                                  b1.reshape(1, -1), W2, b2.reshape(1, -1))
    out_t = _sc_expand(gate_t, env_index.astype(jnp.int32))
    return out_t[:, :, None]
